# baseline (speedup 1.0000x reference)
import functools

import jax
import jax.numpy as jnp
from jax import lax
from jax.experimental import pallas as pl
from jax.experimental.pallas import tpu as pltpu
from jax.experimental.pallas import tpu_sc as plsc

TOP_K = 2
NUM_EXPERTS = 64
D_MODEL = 768
N_TOKENS = 32768

E = NUM_EXPERTS
NC, NS, L = 2, 16, 16
NW = NC * NS
TPW = N_TOKENS // NW
CHUNK = 512
NGROUP = CHUNK // L
NCHUNK = TPW // CHUNK

BLOCK = 1024

NEG_INF = float("-inf")


def _tc_matmul_body(tokens_ref, wg_ref, logits_ref):
    logits_ref[...] = jnp.dot(
        tokens_ref[...], wg_ref[...], preferred_element_type=jnp.float32)


def _tc_matmul(tokens, w_gate):
    return pl.pallas_call(
        _tc_matmul_body,
        grid=(N_TOKENS // BLOCK,),
        in_specs=[
            pl.BlockSpec((BLOCK, D_MODEL), lambda i: (i, 0)),
            pl.BlockSpec((D_MODEL, E), lambda i: (0, 0)),
        ],
        out_specs=pl.BlockSpec((BLOCK, E), lambda i: (i, 0)),
        out_shape=jax.ShapeDtypeStruct((N_TOKENS, E), jnp.float32),
    )(tokens, w_gate)


def _sc_router_body(logits_hbm, sel_hbm, ew_hbm, loadp_hbm,
                    lg_v, ew_v, sel_v, cnt_v, red_v):
    c = lax.axis_index("c")
    s = lax.axis_index("s")
    wid = s * NC + c
    base = wid * TPW

    lane = lax.iota(jnp.int32, L)
    zf = jnp.zeros((L,), jnp.float32)
    zi = jnp.zeros((L,), jnp.int32)
    ones_i = jnp.ones((L,), jnp.int32)

    def zc(i, _):
        cnt_v[pl.ds(i * L, L)] = zi
        return 0
    lax.fori_loop(0, (L * E) // L, zc, 0, unroll=8)

    def do_chunk(ci, _):
        chunk_tok = ci * CHUNK
        pltpu.sync_copy(
            logits_hbm.at[pl.ds((base + chunk_tok) * E, CHUNK * E)], lg_v)

        def zw(i, _):
            ew_v[pl.ds(i * L, L)] = zf
            return 0
        lax.fori_loop(0, (CHUNK * E) // L, zw, 0, unroll=8)

        def do_group(g, _):
            rows = g * L + lane
            rowb = rows * E

            def scan_e(e, carry):
                m1, m2, i1, i2 = carry
                v = plsc.load_gather(lg_v, [rowb + e])
                ev = jnp.full((L,), e, jnp.int32)
                c1 = v > m1
                c2 = v > m2
                nm2 = jnp.where(c1, m1, jnp.where(c2, v, m2))
                ni2 = jnp.where(c1, i1, jnp.where(c2, ev, i2))
                nm1 = jnp.where(c1, v, m1)
                ni1 = jnp.where(c1, ev, i1)
                return nm1, nm2, ni1, ni2

            m1, m2, i1, i2 = lax.fori_loop(
                0, E, scan_e,
                (jnp.full((L,), NEG_INF, jnp.float32),
                 jnp.full((L,), NEG_INF, jnp.float32), zi, zi),
                unroll=8)

            ex = jnp.exp(m2 - m1)
            w1 = 1.0 / (1.0 + ex)
            w2 = ex * w1

            plsc.store_scatter(ew_v, [rowb + i1], w1)
            plsc.store_scatter(ew_v, [rowb + i2], w2)

            selb = (chunk_tok + rows) * 2
            plsc.store_scatter(sel_v, [selb], i1)
            plsc.store_scatter(sel_v, [selb + 1], i2)

            laneb = lane * E
            plsc.addupdate_scatter(cnt_v, [laneb + i1], ones_i)
            plsc.addupdate_scatter(cnt_v, [laneb + i2], ones_i, mask=w2 > 0.0)
            return 0

        lax.fori_loop(0, NGROUP, do_group, 0)

        pltpu.sync_copy(
            ew_v, ew_hbm.at[pl.ds((base + chunk_tok) * E, CHUNK * E)])
        return 0

    lax.fori_loop(0, NCHUNK, do_chunk, 0)

    pltpu.sync_copy(sel_v, sel_hbm.at[pl.ds(base * 2, TPW * 2)])

    for c4 in range(E // L):
        def rsum(r, acc):
            return acc + cnt_v[pl.ds(r * E + c4 * L, L)]
        acc = lax.fori_loop(0, L, rsum, zi, unroll=8)
        red_v[pl.ds(c4 * L, L)] = acc.astype(jnp.float32)

    pltpu.sync_copy(red_v, loadp_hbm.at[pl.ds(wid * E, E)])


def _make_sc_router():
    mesh = plsc.VectorSubcoreMesh(
        core_axis_name="c", subcore_axis_name="s", num_cores=NC)
    return pl.kernel(
        _sc_router_body,
        out_type=[
            jax.ShapeDtypeStruct((N_TOKENS * TOP_K,), jnp.int32),
            jax.ShapeDtypeStruct((N_TOKENS * E,), jnp.float32),
            jax.ShapeDtypeStruct((NW * E,), jnp.float32),
        ],
        mesh=mesh,
        scratch_types=[
            pltpu.VMEM((CHUNK * E,), jnp.float32),
            pltpu.VMEM((CHUNK * E,), jnp.float32),
            pltpu.VMEM((TPW * TOP_K,), jnp.int32),
            pltpu.VMEM((L * E,), jnp.int32),
            pltpu.VMEM((E,), jnp.float32),
        ],
        compiler_params=pltpu.CompilerParams(needs_layout_passes=False),
    )


@jax.jit
def kernel(tokens, w_gate, w_noise):
    del w_noise
    logits = _tc_matmul(tokens, w_gate)
    sel_f, ew_f, loadp = _make_sc_router()(logits.reshape(-1))
    return (
        logits,
        sel_f.reshape(N_TOKENS, TOP_K),
        ew_f.reshape(N_TOKENS, E),
        loadp.reshape(NW, E).sum(axis=0),
    )
